# Initial kernel scaffold; baseline (speedup 1.0000x reference)
import jax
import jax.numpy as jnp
from jax.experimental import pallas as pl


def kernel(x, intent, stat_keys, token_keys, values, exemplar, params, dec_valid_lens, stat_valid_lens, ex_valid_lens):
    raise NotImplementedError("write your pallas kernel here")


# speedup vs baseline: 13.3051x; 13.3051x over previous
import math

import jax
import jax.numpy as jnp
from jax import lax
from jax.experimental import pallas as pl
from jax.experimental.pallas import tpu as pltpu

_B, _Q, _S, _T, _EX = 8, 128, 32, 64, 64
_D, _DI, _DFF, _H = 512, 64, 2048, 8
_DH = _D // _H
_STAT_K, _TOKEN_K = 4, 8
_NEG = -1e6
_F32 = jnp.float32


def _dot(a, b):
    return lax.dot_general(a, b, (((1,), (0,)), ((), ())),
                           preferred_element_type=_F32)


def _dot_t(a, b):
    return lax.dot_general(a, b, (((1,), (1,)), ((), ())),
                           preferred_element_type=_F32)


def _softmax(x):
    m = jnp.max(x, axis=-1, keepdims=True)
    e = jnp.exp(x - m)
    return e / jnp.sum(e, axis=-1, keepdims=True)


def _layer_norm(x, g, b):
    m = jnp.mean(x, axis=-1, keepdims=True)
    c = x - m
    v = jnp.mean(c * c, axis=-1, keepdims=True)
    return c * lax.rsqrt(v + 1e-5) * g + b


def _kth_threshold(s, k, axis):
    t = jnp.max(s, axis=axis, keepdims=True)
    for _ in range(k - 1):
        t = jnp.max(jnp.where(s < t, s, -jnp.inf), axis=axis, keepdims=True)
    return jnp.maximum(t, _NEG)


def _softmax_ax(x, axis):
    m = jnp.max(x, axis=axis, keepdims=True)
    e = jnp.exp(x - m)
    return e / jnp.sum(e, axis=axis, keepdims=True)


def _mha_core(qin, kin, valid, Wq, Wk, Wv, Wo):
    Qp = _dot(qin, Wq)
    Kp = _dot(kin, Wk)
    Vp = _dot(kin, Wv)
    nq, nk = qin.shape[0], kin.shape[0]
    kidx = lax.broadcasted_iota(jnp.int32, (nq, nk), 1)
    mask = kidx < valid
    scale = 1.0 / math.sqrt(_DH)
    outs = []
    for h in range(_H):
        sl = slice(h * _DH, (h + 1) * _DH)
        s = _dot_t(Qp[:, sl], Kp[:, sl]) * scale
        s = jnp.where(mask, s, _NEG)
        outs.append(_dot(_softmax(s), Vp[:, sl]))
    return _dot(jnp.concatenate(outs, axis=-1), Wo)


def _blk_self(dec_ref, x_ref, wq, wk, wv, wo, g1, b1, out_ref):
    b = pl.program_id(0)
    xb = x_ref[0]
    y = _mha_core(xb, xb, dec_ref[b], wq[...], wk[...], wv[...], wo[...])
    out_ref[0] = _layer_norm(xb + y, g1[...], b1[...])


def _blk_selective(stat_ref, q_ref, sk_ref, tk_ref, val_ref,
                   wqs, wqt, wks, wkt, wv, wo, out_ref):
    b = pl.program_id(0)
    qc = q_ref[0]
    scale = 1.0 / math.sqrt(_D)

    qs = _dot(qc, wqs[...])
    ks = _dot(sk_ref[0], wks[...])
    ssT = _dot_t(ks, qs) * scale
    gidx = lax.broadcasted_iota(jnp.int32, (_S, _Q), 0)
    ssT = jnp.where(gidx < stat_ref[b], ssT, _NEG)
    swT = _softmax_ax(
        jnp.where(ssT >= _kth_threshold(ssT, _STAT_K, 0), ssT, _NEG), 0)

    qt = _dot(qc, wqt[...])
    qt2 = _dot_t(qt, wkt[...])
    tscT = _dot_t(tk_ref[0], qt2) * scale
    ts3 = tscT.reshape(_S, _T, _Q)
    tw3 = _softmax_ax(
        jnp.where(ts3 >= _kth_threshold(ts3, _TOKEN_K, 1), ts3, _NEG), 1)

    comb = (swT[:, None, :] * tw3).reshape(_S * _T, _Q)
    ctx = lax.dot_general(comb, val_ref[0], (((0,), (0,)), ((), ())),
                          preferred_element_type=_F32)
    out_ref[0] = _dot(_dot(ctx, wv[...]), wo[...])


def _blk_cross(exv_ref, q_ref, ex_ref, wq, wk, wv, wo, out_ref):
    b = pl.program_id(0)
    out_ref[0] = _mha_core(q_ref[0], ex_ref[0], exv_ref[b],
                           wq[...], wk[...], wv[...], wo[...])


def _blk_tail(x1_ref, sel_ref, exo_ref, gt, w1, b1, w2, b2,
              g2, bb2, g3, bb3, out_ref):
    x1 = x1_ref[0]
    sel = sel_ref[0]
    exo = exo_ref[0]
    gw = gt[...]
    logit = (jnp.sum(sel * gw[:, :_D], axis=-1, keepdims=True)
             + jnp.sum(exo * gw[:, _D:], axis=-1, keepdims=True))
    g = jax.nn.sigmoid(logit)
    x2 = _layer_norm(x1 + g * sel + (1.0 - g) * exo, g2[...], bb2[...])
    h = jnp.maximum(_dot(x2, w1[...]) + b1[...], 0.0)
    ff = _dot(h, w2[...]) + b2[...]
    out_ref[0] = _layer_norm(x2 + ff, g3[...], bb3[...])


def _batched(shape):
    n = len(shape) - 1
    return pl.BlockSpec((1,) + tuple(shape[1:]),
                        lambda b, *_: (b,) + (0,) * n)


def _full(shape):
    n = len(shape)
    return pl.BlockSpec(tuple(shape), lambda b, *_: (0,) * n)


def _call(body, scalar, arrays, out_shape):
    in_specs = [_batched(a.shape) if flag else _full(a.shape)
                for a, flag in arrays]
    grid_spec = pltpu.PrefetchScalarGridSpec(
        num_scalar_prefetch=0 if scalar is None else 1,
        grid=(_B,),
        in_specs=in_specs,
        out_specs=_batched(out_shape),
    )
    args = [a for a, _ in arrays]
    if scalar is not None:
        args = [scalar] + args
    return pl.pallas_call(
        body,
        grid_spec=grid_spec,
        out_shape=jax.ShapeDtypeStruct(out_shape, _F32),
    )(*args)


def kernel(x, intent, stat_keys, token_keys, values, exemplar, params,
           dec_valid_lens, stat_valid_lens, ex_valid_lens):
    P = params
    dec = dec_valid_lens.astype(jnp.int32)
    stv = stat_valid_lens.astype(jnp.int32)
    exv = ex_valid_lens.astype(jnp.int32)
    tk = token_keys.reshape(_B, _S * _T, _D)
    vals = values.reshape(_B, _S * _T, _D)
    r = lambda a, n: a.reshape(1, n)

    x1 = _call(_blk_self, dec,
               [(x, True), (P['ma_Wq'], False), (P['ma_Wk'], False),
                (P['ma_Wv'], False), (P['ma_Wo'], False),
                (r(P['ln1_g'], _D), False), (r(P['ln1_b'], _D), False)],
               (_B, _Q, _D))

    qc = jnp.concatenate([x1, intent], axis=-1)

    sel = _call(_blk_selective, stv,
                [(qc, True), (stat_keys, True), (tk, True), (vals, True),
                 (P['sa_Wqs'], False), (P['sa_Wqt'], False),
                 (P['sa_Wks'], False), (P['sa_Wkt'], False),
                 (P['sa_Wv'], False), (P['sa_Wo'], False)],
                (_B, _Q, _D))

    exo = _call(_blk_cross, exv,
                [(qc, True), (exemplar, True),
                 (P['ca_Wq'], False), (P['ca_Wk'], False),
                 (P['ca_Wv'], False), (P['ca_Wo'], False)],
                (_B, _Q, _D))

    out = _call(_blk_tail, None,
                [(x1, True), (sel, True), (exo, True),
                 (P['gate_W'].reshape(1, 2 * _D), False),
                 (P['ffn_W1'], False), (r(P['ffn_b1'], _DFF), False),
                 (P['ffn_W2'], False), (r(P['ffn_b2'], _D), False),
                 (r(P['ln2_g'], _D), False), (r(P['ln2_b'], _D), False),
                 (r(P['ln3_g'], _D), False), (r(P['ln3_b'], _D), False)],
                (_B, _Q, _D))
    return out
